# baseline (speedup 1.0000x reference)
import functools

import jax
import jax.numpy as jnp
from jax import lax
from jax.experimental import pallas as pl
from jax.experimental.pallas import tpu as pltpu
from jax.experimental.pallas import tpu_sc as plsc

NUM_CORES = 2
NUM_SUBCORES = 16
NW = NUM_CORES * NUM_SUBCORES
CHUNK = 128


def kernel(pos_h, pos_r, pos_t, entity_embds, rel_embds):
    B = pos_h.shape[0]
    D = entity_embds.shape[1]
    b_per_w = B // NW
    n_chunk = b_per_w // CHUNK

    idx_h = pos_h.astype(jnp.int32).reshape(NW, n_chunk, CHUNK)
    idx_r = pos_r.astype(jnp.int32).reshape(NW, n_chunk, CHUNK)
    idx_t = pos_t.astype(jnp.int32).reshape(NW, n_chunk, CHUNK)

    mesh = plsc.VectorSubcoreMesh(
        core_axis_name="c", subcore_axis_name="s",
        num_cores=NUM_CORES, num_subcores=NUM_SUBCORES,
    )

    out = jax.ShapeDtypeStruct((B, D), jnp.float32)

    @functools.partial(
        pl.kernel,
        out_type=(out, out, out),
        mesh=mesh,
        compiler_params=pltpu.CompilerParams(use_tc_tiling_on_sc=False),
        scratch_types=[
            pltpu.VMEM((n_chunk, CHUNK), jnp.int32),
            pltpu.VMEM((n_chunk, CHUNK), jnp.int32),
            pltpu.VMEM((n_chunk, CHUNK), jnp.int32),
            pltpu.VMEM((b_per_w, D), jnp.float32),
            pltpu.VMEM((b_per_w, D), jnp.float32),
            pltpu.VMEM((b_per_w, D), jnp.float32),
            pltpu.SemaphoreType.DMA,
        ],
    )
    def run(ih_hbm, ir_hbm, it_hbm, ent_hbm, rel_hbm,
            oh_hbm, or_hbm, ot_hbm,
            ih_v, ir_v, it_v, rh_v, rr_v, rt_v, sem):
        wid = lax.axis_index("s") * NUM_CORES + lax.axis_index("c")
        base = wid * b_per_w

        pltpu.sync_copy(ih_hbm.at[wid], ih_v)
        pltpu.sync_copy(ir_hbm.at[wid], ir_v)
        pltpu.sync_copy(it_hbm.at[wid], it_v)

        copies = []
        for j in range(n_chunk):
            dst = pl.ds(j * CHUNK, CHUNK)
            copies.append(
                pltpu.async_copy(ent_hbm.at[ih_v.at[j]], rh_v.at[dst], sem))
            copies.append(
                pltpu.async_copy(ent_hbm.at[it_v.at[j]], rt_v.at[dst], sem))
            copies.append(
                pltpu.async_copy(rel_hbm.at[ir_v.at[j]], rr_v.at[dst], sem))
        for c in copies:
            c.wait()

        pltpu.sync_copy(rh_v, oh_hbm.at[pl.ds(base, b_per_w)])
        pltpu.sync_copy(rr_v, or_hbm.at[pl.ds(base, b_per_w)])
        pltpu.sync_copy(rt_v, ot_hbm.at[pl.ds(base, b_per_w)])

    return run(idx_h, idx_r, idx_t, entity_embds, rel_embds)


# speedup vs baseline: 1.5491x; 1.5491x over previous
import functools

import jax
import jax.numpy as jnp
from jax import lax
from jax.experimental import pallas as pl
from jax.experimental.pallas import tpu as pltpu
from jax.experimental.pallas import tpu_sc as plsc

NUM_CORES = 2
NUM_SUBCORES = 16
NW = NUM_CORES * NUM_SUBCORES
LANES = 16
CHUNK = 256


def kernel(pos_h, pos_r, pos_t, entity_embds, rel_embds):
    B = pos_h.shape[0]
    D = entity_embds.shape[1]
    b_per_w = B // NW
    n_chunk = b_per_w // CHUNK

    idx_h = pos_h.astype(jnp.int32)
    idx_r = pos_r.astype(jnp.int32)
    idx_t = pos_t.astype(jnp.int32)

    mesh = plsc.VectorSubcoreMesh(
        core_axis_name="c", subcore_axis_name="s",
        num_cores=NUM_CORES, num_subcores=NUM_SUBCORES,
    )

    out = jax.ShapeDtypeStruct((B, D), jnp.float32)

    @functools.partial(
        pl.kernel,
        out_type=(out, out, out),
        mesh=mesh,
        compiler_params=pltpu.CompilerParams(
            use_tc_tiling_on_sc=True, needs_layout_passes=False),
        scratch_types=[
            pltpu.VMEM((b_per_w,), jnp.int32),
            pltpu.VMEM((b_per_w,), jnp.int32),
            pltpu.VMEM((b_per_w,), jnp.int32),
            pltpu.VMEM((CHUNK, 32), jnp.float32),
            pltpu.VMEM((CHUNK, 32), jnp.float32),
            pltpu.VMEM((CHUNK, 32), jnp.float32),
            pltpu.SemaphoreType.DMA,
            pltpu.SemaphoreType.DMA,
            pltpu.SemaphoreType.DMA,
            pltpu.SemaphoreType.DMA,
        ],
    )
    def run(ih_hbm, ir_hbm, it_hbm, ent_hbm, rel_hbm,
            oh_hbm, or_hbm, ot_hbm,
            ih_v, ir_v, it_v, rows_h, rows_r, rows_t,
            sem_h, sem_r, sem_t, sem_out):
        wid = lax.axis_index("s") * NUM_CORES + lax.axis_index("c")
        base = wid * b_per_w

        pltpu.sync_copy(ih_hbm.at[pl.ds(base, b_per_w)], ih_v)
        pltpu.sync_copy(ir_hbm.at[pl.ds(base, b_per_w)], ir_v)
        pltpu.sync_copy(it_hbm.at[pl.ds(base, b_per_w)], it_v)

        def fire_rows(idx_v, table, rows_v, sem, c):
            def body(g, _):
                vec = idx_v[pl.ds(c * CHUNK + g * LANES, LANES)]
                for l in range(LANES):
                    sel = jnp.where(lax.iota(jnp.int32, LANES) == l, vec, 0)
                    row = lax.reduce_sum(sel, axes=(0,))
                    pltpu.async_copy(
                        table.at[pl.ds(row, 1)],
                        rows_v.at[pl.ds(g * LANES + l, 1)], sem)
                return 0

            lax.fori_loop(0, CHUNK // LANES, body, 0)

        def drain(rows_v, sem):
            pltpu.make_async_copy(
                ent_hbm.at[pl.ds(0, CHUNK)], rows_v, sem).wait()

        out_copies = []
        for c in range(n_chunk):
            fire_rows(ih_v, ent_hbm, rows_h, sem_h, c)
            fire_rows(it_v, ent_hbm, rows_t, sem_t, c)
            fire_rows(ir_v, rel_hbm, rows_r, sem_r, c)
            dst = pl.ds(base + c * CHUNK, CHUNK)
            drain(rows_h, sem_h)
            out_copies.append(
                pltpu.async_copy(rows_h, oh_hbm.at[dst], sem_out))
            drain(rows_t, sem_t)
            out_copies.append(
                pltpu.async_copy(rows_t, ot_hbm.at[dst], sem_out))
            drain(rows_r, sem_r)
            out_copies.append(
                pltpu.async_copy(rows_r, or_hbm.at[dst], sem_out))
            if c + 1 < n_chunk:
                for cp in out_copies:
                    cp.wait()
                out_copies = []
        for cp in out_copies:
            cp.wait()

    return run(idx_h, idx_r, idx_t, entity_embds, rel_embds)
